# Initial kernel scaffold; baseline (speedup 1.0000x reference)
import jax
import jax.numpy as jnp
from jax.experimental import pallas as pl


def kernel(x, router_scale, router_logits, gating_einsum, linear, per_expert_scale):
    raise NotImplementedError("write your pallas kernel here")


# speedup vs baseline: 8.2048x; 8.2048x over previous
import functools

import jax
import jax.numpy as jnp
from jax import lax
from jax.experimental import pallas as pl
from jax.experimental.pallas import tpu as pltpu
from jax.experimental.pallas import tpu_sc as plsc

ROW_TILE = 64


def _router_body(x_ref, rs_ref, rl_ref, perm_ref, pos_ref, off_ref):
    T, D = x_ref.shape
    E = rl_ref.shape[1]
    x = x_ref[...]
    var = jnp.mean(x * x, axis=1, keepdims=True)
    xn = x * lax.rsqrt(var + 1e-6)
    xn = xn * lax.rsqrt(jnp.float32(D)) * rs_ref[...]
    logits = jnp.dot(xn, rl_ref[...], preferred_element_type=jnp.float32)
    expert = jnp.argmax(logits, axis=1).astype(jnp.int32)

    onehot = (expert[:, None] == lax.broadcasted_iota(jnp.int32, (T, E), 1))
    onehot = onehot.astype(jnp.float32)

    counts = jnp.sum(onehot, axis=0, keepdims=True)
    tri = (lax.broadcasted_iota(jnp.int32, (E, E), 0)
           < lax.broadcasted_iota(jnp.int32, (E, E), 1)).astype(jnp.float32)
    offs = jnp.dot(counts, tri, preferred_element_type=jnp.float32)
    ends = offs + counts

    s = onehot
    k = 1
    while k < T:
        s = s + jnp.concatenate(
            [jnp.zeros((k, E), jnp.float32), s[: T - k, :]], axis=0)
        k *= 2
    rank = jnp.sum(s * onehot, axis=1) - 1.0
    seg_base = jnp.sum(onehot * offs, axis=1)
    pos = rank + seg_base

    pos_i = pos.astype(jnp.int32)
    pos_ref[...] = pos_i
    off_ref[...] = jnp.concatenate([offs, ends], axis=0).astype(jnp.int32)

    ids = lax.broadcasted_iota(jnp.int32, (1, T), 1).astype(jnp.float32)
    CH = 256
    for j in range(T // CH):
        sel = (pos_i[:, None]
               == (j * CH + lax.broadcasted_iota(jnp.int32, (1, CH), 1)))
        chunk = jnp.dot(ids, sel.astype(jnp.float32),
                        preferred_element_type=jnp.float32)
        perm_ref[pl.ds(j * CH, CH)] = chunk.reshape(CH).astype(jnp.int32)


def _router(x2d, router_scale, router_logits):
    T, D = x2d.shape
    E = router_logits.shape[1]
    return pl.pallas_call(
        _router_body,
        out_shape=(
            jax.ShapeDtypeStruct((T,), jnp.int32),
            jax.ShapeDtypeStruct((T,), jnp.int32),
            jax.ShapeDtypeStruct((2, E), jnp.int32),
        ),
    )(x2d, router_scale, router_logits)


def _sc_gather_rows(table, idx):
    T, D = table.shape
    B = idx.shape[0]
    info = plsc.get_sparse_core_info()
    nw = info.num_cores * info.num_subcores
    b_per_w = B // nw
    mesh = plsc.VectorSubcoreMesh(core_axis_name="c", subcore_axis_name="s")

    @functools.partial(
        pl.kernel, mesh=mesh,
        out_type=jax.ShapeDtypeStruct((B, D), jnp.float32),
        scratch_types=[
            pltpu.VMEM((b_per_w,), jnp.int32),
            pltpu.VMEM((b_per_w, D), jnp.float32),
            pltpu.SemaphoreType.DMA,
        ],
    )
    def k(table_hbm, idx_hbm, out_hbm, idx_v, rows_v, sem):
        wid = lax.axis_index("s") * info.num_cores + lax.axis_index("c")
        base = wid * b_per_w
        pltpu.sync_copy(idx_hbm.at[pl.ds(base, b_per_w)], idx_v)
        pltpu.async_copy(table_hbm.at[idx_v], rows_v, sem).wait()
        pltpu.sync_copy(rows_v, out_hbm.at[pl.ds(base, b_per_w)])

    return k(table, idx)


def _ffn_body(off_ref, scale_ref, xs_ref, g_ref, l_ref, out_ref):
    e = pl.program_id(0)
    T, D = xs_ref.shape

    @pl.when(e == 0)
    def _():
        out_ref[...] = jnp.zeros_like(out_ref)

    start = off_ref[0, e]
    end = off_ref[1, e]
    t0 = (start // ROW_TILE) * ROW_TILE
    ntiles = lax.select(end > start,
                        (end - t0 + ROW_TILE - 1) // ROW_TILE,
                        jnp.int32(0))
    w0 = g_ref[0, 0]
    w1 = g_ref[0, 1]
    w2 = l_ref[0]
    sc = scale_ref[e]

    def body(i, carry):
        r0 = t0 + i * ROW_TILE
        rows = xs_ref[pl.ds(r0, ROW_TILE), :]
        dn = (((1,), (1,)), ((), ()))
        g0 = lax.dot_general(rows, w0, dn, preferred_element_type=jnp.float32)
        g1 = lax.dot_general(rows, w1, dn, preferred_element_type=jnp.float32)
        act = jax.nn.gelu(g0) * g1
        o = jnp.dot(act, w2, preferred_element_type=jnp.float32)
        ridx = r0 + lax.broadcasted_iota(jnp.int32, (ROW_TILE, 1), 0)
        m = (ridx >= start) & (ridx < end)
        o = jnp.where(m, o * sc, 0.0)
        out_ref[pl.ds(r0, ROW_TILE), :] += o
        return carry

    lax.fori_loop(0, ntiles, body, 0)


def _ffn(x_sorted, offs, gating, linear, scale):
    T, D = x_sorted.shape
    E, _, H, _ = gating.shape
    return pl.pallas_call(
        _ffn_body,
        grid=(E,),
        in_specs=[
            pl.BlockSpec(memory_space=pltpu.SMEM),
            pl.BlockSpec(memory_space=pltpu.SMEM),
            pl.BlockSpec((T, D), lambda e: (0, 0)),
            pl.BlockSpec((1, 2, H, D), lambda e: (e, 0, 0, 0)),
            pl.BlockSpec((1, H, D), lambda e: (e, 0, 0)),
        ],
        out_specs=pl.BlockSpec((T, D), lambda e: (0, 0)),
        out_shape=jax.ShapeDtypeStruct((T, D), jnp.float32),
    )(offs, scale, x_sorted, gating, linear)


def kernel(x, router_scale, router_logits, gating_einsum, linear,
           per_expert_scale):
    B, L, D = x.shape
    x2d = x.reshape(B * L, D)
    perm, pos, offs = _router(x2d, router_scale, router_logits)
    x_sorted = _sc_gather_rows(x2d, perm)
    out_sorted = _ffn(x_sorted, offs, gating_einsum, linear, per_expert_scale)
    out = _sc_gather_rows(out_sorted, pos)
    return out.reshape(B, L, D)


# speedup vs baseline: 8.2561x; 1.0062x over previous
import functools

import jax
import jax.numpy as jnp
from jax import lax
from jax.experimental import pallas as pl
from jax.experimental.pallas import tpu as pltpu
from jax.experimental.pallas import tpu_sc as plsc

ROW_TILE = 64


def _router_body(x_ref, rs_ref, rl_ref, perm_ref, pos_ref, off_ref):
    T, D = x_ref.shape
    E = rl_ref.shape[1]
    x = x_ref[...]
    var = jnp.mean(x * x, axis=1, keepdims=True)
    xn = x * lax.rsqrt(var + 1e-6)
    xn = xn * lax.rsqrt(jnp.float32(D)) * rs_ref[...]
    logits = jnp.dot(xn, rl_ref[...], preferred_element_type=jnp.float32)
    expert = jnp.argmax(logits, axis=1).astype(jnp.int32)

    onehot = (expert[:, None] == lax.broadcasted_iota(jnp.int32, (T, E), 1))
    onehot = onehot.astype(jnp.float32)

    counts = jnp.sum(onehot, axis=0, keepdims=True)
    tri = (lax.broadcasted_iota(jnp.int32, (E, E), 0)
           < lax.broadcasted_iota(jnp.int32, (E, E), 1)).astype(jnp.float32)
    offs = jnp.dot(counts, tri, preferred_element_type=jnp.float32)
    ends = offs + counts

    s = onehot
    k = 1
    while k < T:
        s = s + jnp.concatenate(
            [jnp.zeros((k, E), jnp.float32), s[: T - k, :]], axis=0)
        k *= 2
    rank = jnp.sum(s * onehot, axis=1) - 1.0
    seg_base = jnp.sum(onehot * offs, axis=1)
    pos = rank + seg_base

    pos_i = pos.astype(jnp.int32)
    pos_ref[...] = pos_i
    off_ref[...] = jnp.concatenate([offs, ends], axis=0).astype(jnp.int32)

    ids = lax.broadcasted_iota(jnp.int32, (1, T), 1).astype(jnp.float32)
    CH = 256
    for j in range(T // CH):
        sel = (pos_i[:, None]
               == (j * CH + lax.broadcasted_iota(jnp.int32, (1, CH), 1)))
        chunk = jnp.dot(ids, sel.astype(jnp.float32),
                        preferred_element_type=jnp.float32)
        perm_ref[pl.ds(j * CH, CH)] = chunk.reshape(CH).astype(jnp.int32)


def _router(x2d, router_scale, router_logits):
    T, D = x2d.shape
    E = router_logits.shape[1]
    return pl.pallas_call(
        _router_body,
        out_shape=(
            jax.ShapeDtypeStruct((T,), jnp.int32),
            jax.ShapeDtypeStruct((T,), jnp.int32),
            jax.ShapeDtypeStruct((2, E), jnp.int32),
        ),
    )(x2d, router_scale, router_logits)


def _sc_gather_rows(table, idx):
    T, D = table.shape
    B = idx.shape[0]
    info = plsc.get_sparse_core_info()
    nw = info.num_cores * info.num_subcores
    b_per_w = B // nw
    mesh = plsc.VectorSubcoreMesh(core_axis_name="c", subcore_axis_name="s")

    @functools.partial(
        pl.kernel, mesh=mesh,
        out_type=jax.ShapeDtypeStruct((B, D), jnp.float32),
        scratch_types=[
            pltpu.VMEM((b_per_w,), jnp.int32),
            pltpu.VMEM((b_per_w, D), jnp.float32),
            pltpu.SemaphoreType.DMA,
        ],
    )
    def k(table_hbm, idx_hbm, out_hbm, idx_v, rows_v, sem):
        wid = lax.axis_index("s") * info.num_cores + lax.axis_index("c")
        base = wid * b_per_w
        pltpu.sync_copy(idx_hbm.at[pl.ds(base, b_per_w)], idx_v)
        pltpu.async_copy(table_hbm.at[idx_v], rows_v, sem).wait()
        pltpu.sync_copy(rows_v, out_hbm.at[pl.ds(base, b_per_w)])

    return k(table, idx)


def _ffn_body(off_ref, scale_ref, xs_ref, g_ref, l_ref, out_ref):
    e = pl.program_id(0)
    T, D = xs_ref.shape
    start = off_ref[0, e]
    end = off_ref[1, e]
    t0 = (start // ROW_TILE) * ROW_TILE
    ntiles = lax.select(end > start,
                        (end - t0 + ROW_TILE - 1) // ROW_TILE,
                        jnp.int32(0))
    w0 = g_ref[0, 0]
    w1 = g_ref[0, 1]
    w2 = l_ref[0]
    sc = scale_ref[e]

    def body(i, carry):
        r0 = t0 + i * ROW_TILE
        rows = xs_ref[pl.ds(r0, ROW_TILE), :]
        dn = (((1,), (1,)), ((), ()))
        g0 = lax.dot_general(rows, w0, dn, preferred_element_type=jnp.float32)
        g1 = lax.dot_general(rows, w1, dn, preferred_element_type=jnp.float32)
        act = jax.nn.gelu(g0) * g1
        o = jnp.dot(act, w2, preferred_element_type=jnp.float32)
        ridx = r0 + lax.broadcasted_iota(jnp.int32, (ROW_TILE, 1), 0)
        m = (ridx >= start) & (ridx < end)
        out_ref[pl.ds(r0, ROW_TILE), :] = jnp.where(
            m, o * sc, out_ref[pl.ds(r0, ROW_TILE), :])
        return carry

    lax.fori_loop(0, ntiles, body, 0)


def _ffn(x_sorted, offs, gating, linear, scale):
    T, D = x_sorted.shape
    E, _, H, _ = gating.shape
    return pl.pallas_call(
        _ffn_body,
        grid=(E,),
        in_specs=[
            pl.BlockSpec(memory_space=pltpu.SMEM),
            pl.BlockSpec(memory_space=pltpu.SMEM),
            pl.BlockSpec((T, D), lambda e: (0, 0)),
            pl.BlockSpec((1, 2, H, D), lambda e: (e, 0, 0, 0)),
            pl.BlockSpec((1, H, D), lambda e: (e, 0, 0)),
        ],
        out_specs=pl.BlockSpec((T, D), lambda e: (0, 0)),
        out_shape=jax.ShapeDtypeStruct((T, D), jnp.float32),
    )(offs, scale, x_sorted, gating, linear)


def kernel(x, router_scale, router_logits, gating_einsum, linear,
           per_expert_scale):
    B, L, D = x.shape
    x2d = x.reshape(B * L, D)
    perm, pos, offs = _router(x2d, router_scale, router_logits)
    x_sorted = _sc_gather_rows(x2d, perm)
    out_sorted = _ffn(x_sorted, offs, gating_einsum, linear, per_expert_scale)
    out = _sc_gather_rows(out_sorted, pos)
    return out.reshape(B, L, D)
